# baseline (speedup 1.0000x reference)
import functools

import jax
import jax.numpy as jnp
from jax import lax
from jax.experimental import pallas as pl
from jax.experimental.pallas import tpu as pltpu
from jax.experimental.pallas import tpu_sc as plsc

N = 10000
E = 320000
NC = 2
NS = 16
NW = NC * NS
CK = 128
NCHUNK = 80
EPW = NCHUNK * CK
RPW = E // NW
PPW = EPW - RPW
PAD_ROWS_PER_W = 16
ACC_ROWS = N + NS * PAD_ROWS_PER_W
ROWS_PER_TILE = 624


def _seg_sc_kernel(d):
  mesh = plsc.VectorSubcoreMesh(core_axis_name="c", subcore_axis_name="s")

  @functools.partial(
      pl.kernel,
      out_type=jax.ShapeDtypeStruct((NC, N, d), jnp.float32),
      mesh=mesh,
      scratch_types=[
          pltpu.VMEM((NCHUNK, CK), jnp.int32),
          pltpu.VMEM((NCHUNK, CK), jnp.int32),
          pltpu.VMEM((CK, d), jnp.float32),
          pltpu.VMEM_SHARED((ACC_ROWS, d), jnp.float32),
          pltpu.SemaphoreType.DMA,
      ],
  )
  def k(t_hbm, src_hbm, dst_hbm, out_hbm, src_v, dst_v, rows_v, acc, gsem):
    c = lax.axis_index("c")
    s = lax.axis_index("s")
    w = c * NS + s

    pltpu.sync_copy(t_hbm.at[pl.ds(s * ROWS_PER_TILE, ROWS_PER_TILE)],
                    acc.at[pl.ds(s * ROWS_PER_TILE, ROWS_PER_TILE)])

    @pl.when(s == NS - 1)
    def _():
      pltpu.sync_copy(t_hbm.at[pl.ds(NS * ROWS_PER_TILE, N - NS * ROWS_PER_TILE)],
                      acc.at[pl.ds(NS * ROWS_PER_TILE, N - NS * ROWS_PER_TILE)])

    pltpu.sync_copy(src_hbm.at[w], src_v)
    pltpu.sync_copy(dst_hbm.at[w], dst_v)
    plsc.subcore_barrier()

    def body(j, carry):
      pltpu.async_copy(t_hbm.at[src_v.at[j]], rows_v, gsem).wait()
      pltpu.sync_copy(rows_v, acc.at[dst_v.at[j]], add=True)
      return carry

    lax.fori_loop(0, NCHUNK, body, 0)
    plsc.subcore_barrier()

    pltpu.sync_copy(acc.at[pl.ds(s * ROWS_PER_TILE, ROWS_PER_TILE)],
                    out_hbm.at[c].at[pl.ds(s * ROWS_PER_TILE, ROWS_PER_TILE)])

    @pl.when(s == NS - 1)
    def _():
      pltpu.sync_copy(acc.at[pl.ds(NS * ROWS_PER_TILE, N - NS * ROWS_PER_TILE)],
                      out_hbm.at[c].at[pl.ds(NS * ROWS_PER_TILE,
                                             N - NS * ROWS_PER_TILE)])

  return k


_BR = 2000


def _mm_kernel(x_ref, w_ref, o_ref):
  o_ref[...] = jnp.dot(x_ref[...], w_ref[...],
                       preferred_element_type=jnp.float32)


def _mm(x, w):
  n, d_in = x.shape
  d_out = w.shape[1]
  return pl.pallas_call(
      _mm_kernel,
      grid=(n // _BR,),
      in_specs=[
          pl.BlockSpec((_BR, d_in), lambda i: (i, 0)),
          pl.BlockSpec((d_in, d_out), lambda i: (0, 0)),
      ],
      out_specs=pl.BlockSpec((_BR, d_out), lambda i: (i, 0)),
      out_shape=jax.ShapeDtypeStruct((n, d_out), jnp.float32),
  )(x, w)


def _combine_mm_kernel(p0_ref, p1_ref, t_ref, b_ref, w_ref, o_ref):
  h = jnp.maximum(p0_ref[...] + p1_ref[...] - t_ref[...] + b_ref[...], 0.0)
  o_ref[...] = jnp.dot(h, w_ref[...], preferred_element_type=jnp.float32)


def _combine_mm(p, t, b, w):
  n, d_in = t.shape
  d_out = w.shape[1]
  return pl.pallas_call(
      _combine_mm_kernel,
      grid=(n // _BR,),
      in_specs=[
          pl.BlockSpec((_BR, d_in), lambda i: (i, 0)),
          pl.BlockSpec((_BR, d_in), lambda i: (i, 0)),
          pl.BlockSpec((_BR, d_in), lambda i: (i, 0)),
          pl.BlockSpec((1, d_in), lambda i: (0, 0)),
          pl.BlockSpec((d_in, d_out), lambda i: (0, 0)),
      ],
      out_specs=pl.BlockSpec((_BR, d_out), lambda i: (i, 0)),
      out_shape=jax.ShapeDtypeStruct((n, d_out), jnp.float32),
  )(p[0], p[1], t, b.reshape(1, d_in), w)


def _final_mm_kernel(p0_ref, p1_ref, t_ref, b_ref, w_ref, o_ref):
  z = p0_ref[...] + p1_ref[...] - t_ref[...]
  o_ref[...] = jnp.maximum(
      jnp.dot(z, w_ref[...], preferred_element_type=jnp.float32) + b_ref[...],
      0.0)


def _final_mm(p, t, b, w):
  n, d_in = t.shape
  d_out = w.shape[1]
  return pl.pallas_call(
      _final_mm_kernel,
      grid=(n // _BR,),
      in_specs=[
          pl.BlockSpec((_BR, d_in), lambda i: (i, 0)),
          pl.BlockSpec((_BR, d_in), lambda i: (i, 0)),
          pl.BlockSpec((_BR, d_in), lambda i: (i, 0)),
          pl.BlockSpec((1, d_out), lambda i: (0, 0)),
          pl.BlockSpec((d_in, d_out), lambda i: (0, 0)),
      ],
      out_specs=pl.BlockSpec((_BR, d_out), lambda i: (i, 0)),
      out_shape=jax.ShapeDtypeStruct((n, d_out), jnp.float32),
  )(p[0], p[1], t, b.reshape(1, d_out), w)


def _combine_kernel(p0_ref, p1_ref, t_ref, b_ref, o_ref):
  o_ref[...] = jnp.maximum(
      p0_ref[...] + p1_ref[...] - t_ref[...] + b_ref[...], 0.0)


def _combine(p, t, b):
  n, d = t.shape
  return pl.pallas_call(
      _combine_kernel,
      grid=(n // _BR,),
      in_specs=[
          pl.BlockSpec((_BR, d), lambda i: (i, 0)),
          pl.BlockSpec((_BR, d), lambda i: (i, 0)),
          pl.BlockSpec((_BR, d), lambda i: (i, 0)),
          pl.BlockSpec((1, d), lambda i: (0, 0)),
      ],
      out_specs=pl.BlockSpec((_BR, d), lambda i: (i, 0)),
      out_shape=jax.ShapeDtypeStruct((n, d), jnp.float32),
  )(p[0], p[1], t, b.reshape(1, d))


def kernel(x, edge_index, W1, b1, W2, b2, W3, b3):
  src = edge_index[0].astype(jnp.int32)
  dst = edge_index[1].astype(jnp.int32)
  w_in_core = jnp.arange(NW, dtype=jnp.int32)[:, None] % NS
  src_pad = jnp.zeros((NW, PPW), jnp.int32)
  dst_pad = (N + w_in_core * PAD_ROWS_PER_W
             + (jnp.arange(PPW, dtype=jnp.int32)[None, :] % PAD_ROWS_PER_W))
  src3 = jnp.concatenate([src.reshape(NW, RPW), src_pad], axis=1).reshape(
      NW, NCHUNK, CK)
  dst3 = jnp.concatenate([dst.reshape(NW, RPW), dst_pad], axis=1).reshape(
      NW, NCHUNK, CK)

  seg128 = _seg_sc_kernel(128)

  t1 = _mm(x, W1)
  p1 = seg128(t1, src3, dst3)
  t2 = _combine_mm(p1, t1, b1, W2)
  p2 = seg128(t2, src3, dst3)
  h2 = _combine(p2, t2, b2)
  p3 = seg128(h2, src3, dst3)
  return _final_mm(p3, h2, b3, W3)


# speedup vs baseline: 2.3441x; 2.3441x over previous
import functools

import jax
import jax.numpy as jnp
from jax import lax
from jax.experimental import pallas as pl
from jax.experimental.pallas import tpu as pltpu
from jax.experimental.pallas import tpu_sc as plsc

N = 10000
E = 320000
NC = 2
NS = 16
NW = NC * NS
CK = 128
NCHUNK = 80
EPW = NCHUNK * CK
RPW = E // NW
PPW = EPW - RPW
PAD_ROWS_PER_W = 16
ACC_ROWS = N + NS * PAD_ROWS_PER_W
ROWS_PER_TILE = 624


def _seg_sc_kernel(d):
  mesh = plsc.VectorSubcoreMesh(core_axis_name="c", subcore_axis_name="s")

  @functools.partial(
      pl.kernel,
      out_type=jax.ShapeDtypeStruct((NC, N, d), jnp.float32),
      mesh=mesh,
      scratch_types=[
          pltpu.VMEM((NCHUNK, CK), jnp.int32),
          pltpu.VMEM((NCHUNK, CK), jnp.int32),
          pltpu.VMEM((CK, d), jnp.float32),
          pltpu.VMEM_SHARED((ACC_ROWS, d), jnp.float32),
          pltpu.SemaphoreType.DMA,
      ],
  )
  def k(t_hbm, src_hbm, dst_hbm, out_hbm, src_v, dst_v, rows_v, acc, gsem):
    c = lax.axis_index("c")
    s = lax.axis_index("s")
    w = c * NS + s

    pltpu.sync_copy(t_hbm.at[pl.ds(s * ROWS_PER_TILE, ROWS_PER_TILE)],
                    acc.at[pl.ds(s * ROWS_PER_TILE, ROWS_PER_TILE)])

    @pl.when(s == NS - 1)
    def _():
      pltpu.sync_copy(t_hbm.at[pl.ds(NS * ROWS_PER_TILE, N - NS * ROWS_PER_TILE)],
                      acc.at[pl.ds(NS * ROWS_PER_TILE, N - NS * ROWS_PER_TILE)])

    pltpu.sync_copy(src_hbm.at[w], src_v)
    pltpu.sync_copy(dst_hbm.at[w], dst_v)
    plsc.subcore_barrier()

    def body(j, carry):
      pltpu.async_copy(t_hbm.at[src_v.at[j]], rows_v, gsem).wait()
      pltpu.sync_copy(rows_v, acc.at[dst_v.at[j]], add=True)
      return carry

    lax.fori_loop(0, NCHUNK, body, 0)
    plsc.subcore_barrier()

    pltpu.sync_copy(acc.at[pl.ds(s * ROWS_PER_TILE, ROWS_PER_TILE)],
                    out_hbm.at[c].at[pl.ds(s * ROWS_PER_TILE, ROWS_PER_TILE)])

    @pl.when(s == NS - 1)
    def _():
      pltpu.sync_copy(acc.at[pl.ds(NS * ROWS_PER_TILE, N - NS * ROWS_PER_TILE)],
                      out_hbm.at[c].at[pl.ds(NS * ROWS_PER_TILE,
                                             N - NS * ROWS_PER_TILE)])

  return k


_BR = 2000


def _mm_kernel(x_ref, w_ref, o_ref):
  o_ref[...] = jnp.dot(x_ref[...], w_ref[...],
                       preferred_element_type=jnp.float32)


def _mm(x, w):
  n, d_in = x.shape
  d_out = w.shape[1]
  return pl.pallas_call(
      _mm_kernel,
      grid=(n // _BR,),
      in_specs=[
          pl.BlockSpec((_BR, d_in), lambda i: (i, 0)),
          pl.BlockSpec((d_in, d_out), lambda i: (0, 0)),
      ],
      out_specs=pl.BlockSpec((_BR, d_out), lambda i: (i, 0)),
      out_shape=jax.ShapeDtypeStruct((n, d_out), jnp.float32),
  )(x, w)


def _combine_mm_kernel(p0_ref, p1_ref, t_ref, b_ref, w_ref, o_ref):
  h = jnp.maximum(p0_ref[...] + p1_ref[...] - t_ref[...] + b_ref[...], 0.0)
  o_ref[...] = jnp.dot(h, w_ref[...], preferred_element_type=jnp.float32)


def _combine_mm(p, t, b, w):
  n, d_in = t.shape
  d_out = w.shape[1]
  return pl.pallas_call(
      _combine_mm_kernel,
      grid=(n // _BR,),
      in_specs=[
          pl.BlockSpec((_BR, d_in), lambda i: (i, 0)),
          pl.BlockSpec((_BR, d_in), lambda i: (i, 0)),
          pl.BlockSpec((_BR, d_in), lambda i: (i, 0)),
          pl.BlockSpec((1, d_in), lambda i: (0, 0)),
          pl.BlockSpec((d_in, d_out), lambda i: (0, 0)),
      ],
      out_specs=pl.BlockSpec((_BR, d_out), lambda i: (i, 0)),
      out_shape=jax.ShapeDtypeStruct((n, d_out), jnp.float32),
  )(p[0], p[1], t, b.reshape(1, d_in), w)


def _final_mm_kernel(p0_ref, p1_ref, t_ref, b_ref, w_ref, o_ref):
  z = p0_ref[...] + p1_ref[...] - t_ref[...]
  o_ref[...] = jnp.maximum(
      jnp.dot(z, w_ref[...], preferred_element_type=jnp.float32) + b_ref[...],
      0.0)


def _final_mm(p, t, b, w):
  n, d_in = t.shape
  d_out = w.shape[1]
  return pl.pallas_call(
      _final_mm_kernel,
      grid=(n // _BR,),
      in_specs=[
          pl.BlockSpec((_BR, d_in), lambda i: (i, 0)),
          pl.BlockSpec((_BR, d_in), lambda i: (i, 0)),
          pl.BlockSpec((_BR, d_in), lambda i: (i, 0)),
          pl.BlockSpec((1, d_out), lambda i: (0, 0)),
          pl.BlockSpec((d_in, d_out), lambda i: (0, 0)),
      ],
      out_specs=pl.BlockSpec((_BR, d_out), lambda i: (i, 0)),
      out_shape=jax.ShapeDtypeStruct((n, d_out), jnp.float32),
  )(p[0], p[1], t, b.reshape(1, d_out), w)


def _combine_kernel(p0_ref, p1_ref, t_ref, b_ref, o_ref):
  o_ref[...] = jnp.maximum(
      p0_ref[...] + p1_ref[...] - t_ref[...] + b_ref[...], 0.0)


def _combine(p, t, b):
  n, d = t.shape
  return pl.pallas_call(
      _combine_kernel,
      grid=(n // _BR,),
      in_specs=[
          pl.BlockSpec((_BR, d), lambda i: (i, 0)),
          pl.BlockSpec((_BR, d), lambda i: (i, 0)),
          pl.BlockSpec((_BR, d), lambda i: (i, 0)),
          pl.BlockSpec((1, d), lambda i: (0, 0)),
      ],
      out_specs=pl.BlockSpec((_BR, d), lambda i: (i, 0)),
      out_shape=jax.ShapeDtypeStruct((n, d), jnp.float32),
  )(p[0], p[1], t, b.reshape(1, d))


def kernel(x, edge_index, W1, b1, W2, b2, W3, b3):
  src = edge_index[0].astype(jnp.int32)
  dst = edge_index[1].astype(jnp.int32)
  w_in_core = jnp.arange(NW, dtype=jnp.int32)[:, None] % NS
  src_pad = ((jnp.arange(NW, dtype=jnp.int32)[:, None] * 313
              + jnp.arange(PPW, dtype=jnp.int32)[None, :] * 37) % N)
  dst_pad = (N + w_in_core * PAD_ROWS_PER_W
             + (jnp.arange(PPW, dtype=jnp.int32)[None, :] % PAD_ROWS_PER_W))
  src3 = jnp.concatenate([src.reshape(NW, RPW), src_pad], axis=1).reshape(
      NW, NCHUNK, CK)
  dst3 = jnp.concatenate([dst.reshape(NW, RPW), dst_pad], axis=1).reshape(
      NW, NCHUNK, CK)

  seg128 = _seg_sc_kernel(128)

  t1 = _mm(x, W1)
  p1 = seg128(t1, src3, dst3)
  t2 = _combine_mm(p1, t1, b1, W2)
  p2 = seg128(t2, src3, dst3)
  h2 = _combine(p2, t2, b2)
  p3 = seg128(h2, src3, dst3)
  return _final_mm(p3, h2, b3, W3)
